# baseline (speedup 1.0000x reference)
import jax
import jax.numpy as jnp
from jax import lax
from jax.experimental import pallas as pl
from jax.experimental.pallas import tpu as pltpu
from jax.experimental.pallas import tpu_sc as plsc

N_FEATURES = 26
INPUT_DIM = 33
OUT_DIM = 32
BATCH = 16384

NC, NS, L = 2, 16, 16
NW = NC * NS
TOTAL = BATCH * N_FEATURES
PER_W = TOTAL // NW
TAB_WORDS = N_FEATURES * INPUT_DIM * OUT_DIM
CHUNK = 1024
N_CHUNKS = PER_W // CHUNK
GROUPS = CHUNK // L
OFF_LEN = 208


def _embed_body(idx_hbm, off_hbm, tab_hbm, out_hbm,
                idx_v, off_v, tab_v, rows0, rows1, idx_s, sw0, sw1):
    wid = lax.axis_index("s") * NC + lax.axis_index("c")
    wbase = wid * PER_W
    pltpu.sync_copy(off_hbm, off_v)
    pltpu.sync_copy(tab_hbm, tab_v)
    pltpu.sync_copy(idx_hbm.at[pl.ds(wbase, PER_W)], idx_v)

    @plsc.parallel_loop(0, PER_W // L)
    def _precompute(i):
        off = off_v[pl.ds((i % (OFF_LEN // L)) * L, L)]
        idx_v[pl.ds(i * L, L)] = idx_v[pl.ds(i * L, L)] * OUT_DIM + off

    def chunk_compute(c, buf, idx_s):
        @plsc.parallel_loop(0, GROUPS, unroll=4)
        def _group(g):
            a16 = idx_v[pl.ds(c * CHUNK + g * L, L)]
            for k in range(L):
                a = a16[k]
                base = (g * L + k) * OUT_DIM
                buf[pl.ds(base, L)] = tab_v[pl.ds(a, L)]
                buf[pl.ds(base + L, L)] = tab_v[pl.ds(a + L, L)]

    bufs = (rows0, rows1)
    wsems = (sw0, sw1)
    pend_w = [None, None]

    for c in range(N_CHUNKS):
        b = c % 2
        chunk_compute(c, bufs[b], idx_s)

    wr = pltpu.make_async_copy(
        bufs[0],
        out_hbm.at[pl.ds(wbase * OUT_DIM, CHUNK * OUT_DIM)],
        wsems[0],
    )
    wr.start()
    wr.wait()


def kernel(inputs, tables):
    idx_flat = inputs.reshape(TOTAL)
    tab_flat = tables.reshape(TAB_WORDS)
    off = jnp.tile(
        jnp.arange(N_FEATURES, dtype=jnp.int32) * (INPUT_DIM * OUT_DIM),
        OFF_LEN // N_FEATURES,
    )

    run = pl.kernel(
        _embed_body,
        out_type=jax.ShapeDtypeStruct((TOTAL * OUT_DIM,), jnp.float32),
        mesh=plsc.VectorSubcoreMesh(core_axis_name="c", subcore_axis_name="s"),
        scratch_types=[
            pltpu.VMEM((PER_W,), jnp.int32),
            pltpu.VMEM((OFF_LEN,), jnp.int32),
            pltpu.VMEM((TAB_WORDS,), jnp.float32),
            pltpu.VMEM((CHUNK * OUT_DIM,), jnp.float32),
            pltpu.VMEM((CHUNK * OUT_DIM,), jnp.float32),
            pltpu.SMEM((CHUNK,), jnp.int32),
            pltpu.SemaphoreType.DMA,
            pltpu.SemaphoreType.DMA,
        ],
        compiler_params=pltpu.CompilerParams(
            use_tc_tiling_on_sc=False,
            needs_layout_passes=False,
            disable_bounds_checks=True,
        ),
    )
    out = run(idx_flat, off, tab_flat)
    return out.reshape(BATCH, N_FEATURES * OUT_DIM)


# speedup vs baseline: 1.0297x; 1.0297x over previous
import jax
import jax.numpy as jnp
from jax import lax
from jax.experimental import pallas as pl
from jax.experimental.pallas import tpu as pltpu
from jax.experimental.pallas import tpu_sc as plsc

N_FEATURES = 26
INPUT_DIM = 33
OUT_DIM = 32
BATCH = 16384

NC, NS, L = 2, 16, 16
NW = NC * NS
TOTAL = BATCH * N_FEATURES
PER_W = TOTAL // NW
G = 128
N_GROUPS = PER_W // G
CHUNK = 1664
NG = CHUNK // G
N_CHUNKS = PER_W // CHUNK
OFF_LEN = 208


def _embed_body(idx_hbm, off_hbm, tab_hbm, out_hbm,
                idx_v, off_v, sp_tab, rows0, rows1, sg0, sg1, sw0, sw1):
    cid = lax.axis_index("c")
    sid = lax.axis_index("s")
    wid = sid * NC + cid
    wbase = wid * PER_W

    @pl.when(sid == 0)
    def _stage():
        pltpu.sync_copy(tab_hbm, sp_tab)

    pltpu.sync_copy(off_hbm, off_v)
    pltpu.sync_copy(idx_hbm.at[pl.ds(wbase // G, N_GROUPS)], idx_v)

    @plsc.parallel_loop(0, PER_W // L)
    def _precompute(i):
        r = i // (G // L)
        col = (i % (G // L)) * L
        off = off_v[pl.ds((i % (OFF_LEN // L)) * L, L)]
        idx_v[r, pl.ds(col, L)] = idx_v[r, pl.ds(col, L)] + off

    plsc.subcore_barrier()

    bufs = (rows0, rows1)
    gsems = (sg0, sg1)
    wsems = (sw0, sw1)
    pend_g = [None, None]
    pend_w = [None, None]

    for c in range(N_CHUNKS + 1):
        if c < N_CHUNKS:
            b = c % 2
            if pend_w[b] is not None:
                pend_w[b].wait()
            gs = []
            for g in range(NG):
                cp = pltpu.make_async_copy(
                    sp_tab.at[idx_v.at[c * NG + g]],
                    bufs[b].at[pl.ds(g * G, G)],
                    gsems[b],
                )
                cp.start()
                gs.append(cp)
            pend_g[b] = gs
        if c >= 1:
            b2 = (c - 1) % 2
            for cp in pend_g[b2]:
                cp.wait()
            wr = pltpu.make_async_copy(
                bufs[b2],
                out_hbm.at[pl.ds(wbase + (c - 1) * CHUNK, CHUNK)],
                wsems[b2],
            )
            wr.start()
            pend_w[b2] = wr

    pend_w[(N_CHUNKS - 1) % 2].wait()


def kernel(inputs, tables):
    idx_flat = inputs.reshape(TOTAL // G, G)
    tab_flat = tables.reshape(N_FEATURES * INPUT_DIM, OUT_DIM)
    off = jnp.tile(
        jnp.arange(N_FEATURES, dtype=jnp.int32) * INPUT_DIM,
        OFF_LEN // N_FEATURES,
    )

    run = pl.kernel(
        _embed_body,
        out_type=jax.ShapeDtypeStruct((TOTAL, OUT_DIM), jnp.float32),
        mesh=plsc.VectorSubcoreMesh(core_axis_name="c", subcore_axis_name="s"),
        scratch_types=[
            pltpu.VMEM((N_GROUPS, G), jnp.int32),
            pltpu.VMEM((OFF_LEN,), jnp.int32),
            pltpu.VMEM_SHARED((N_FEATURES * INPUT_DIM, OUT_DIM), jnp.float32),
            pltpu.VMEM((CHUNK, OUT_DIM), jnp.float32),
            pltpu.VMEM((CHUNK, OUT_DIM), jnp.float32),
            pltpu.SemaphoreType.DMA,
            pltpu.SemaphoreType.DMA,
            pltpu.SemaphoreType.DMA,
            pltpu.SemaphoreType.DMA,
        ],
        compiler_params=pltpu.CompilerParams(
            use_tc_tiling_on_sc=False,
            needs_layout_passes=False,
            disable_bounds_checks=True,
        ),
    )
    out = run(idx_flat, off, tab_flat)
    return out.reshape(BATCH, N_FEATURES * OUT_DIM)
